# Initial kernel scaffold; baseline (speedup 1.0000x reference)
import jax
import jax.numpy as jnp
from jax.experimental import pallas as pl


def kernel(x, W, b):
    raise NotImplementedError("write your pallas kernel here")


# speedup vs baseline: 11.3617x; 11.3617x over previous
import functools

import jax
import jax.numpy as jnp
from jax.experimental import pallas as pl

_OLD_A = 2048
_CUR_A = 4096
_TB = 1024


def _mm_dup_kernel(x_ref, w_ref, b_ref, o_ref):
    acc = jnp.dot(x_ref[...], w_ref[...], preferred_element_type=jnp.float32)
    acc = acc + b_ref[...]
    o_ref[:, :_OLD_A] = acc
    o_ref[:, _OLD_A:] = acc


@jax.jit
def kernel(x, W, b):
    batch, d_in = x.shape
    old_a = W.shape[1]
    b2 = b.reshape(1, old_a)
    grid = (batch // _TB,)
    out = pl.pallas_call(
        _mm_dup_kernel,
        grid=grid,
        in_specs=[
            pl.BlockSpec((_TB, d_in), lambda i: (i, 0)),
            pl.BlockSpec((d_in, old_a), lambda i: (0, 0)),
            pl.BlockSpec((1, old_a), lambda i: (0, 0)),
        ],
        out_specs=pl.BlockSpec((_TB, 2 * old_a), lambda i: (i, 0)),
        out_shape=jax.ShapeDtypeStruct((batch, 2 * old_a), jnp.float32),
    )(x, W, b2)
    return out
